# baseline (speedup 1.0000x reference)
import jax
import jax.numpy as jnp
from jax import lax
from jax.experimental import pallas as pl
from jax.experimental.pallas import tpu as pltpu
from jax.experimental.pallas import tpu_sc as plsc

B, D, NG, C = 64, 8732, 20, 81
DP = 8832
R = DP // 128
PR = B * D // 128
THR = 0.5
NW = 32
NSL = 137
CHUNK = NSL * 128
NPAD = NW * CHUNK
NBINS = 4096
BIN_SHIFT = 19


def _match_body(dbt_ref, plt_ref, g_ref, mk_ref, ml_ref, part_ref):
    dl = dbt_ref[0]
    dt = dbt_ref[1]
    dr = dbt_ref[2]
    db = dbt_ref[3]
    area_d = (dr - dl) * (db - dt)

    best = None
    bl = bt = br = bb = blab = None
    for j in range(NG):
        gl = g_ref[0, 0, 0, j]
        gtp = g_ref[0, 0, 1, j]
        gr = g_ref[0, 0, 2, j]
        gb = g_ref[0, 0, 3, j]
        lab = g_ref[0, 0, 4, j]
        area_g = (gr - gl) * (gb - gtp)
        iw = jnp.clip(jnp.minimum(dr, gr) - jnp.maximum(dl, gl), 0.0)
        ih = jnp.clip(jnp.minimum(db, gb) - jnp.maximum(dt, gtp), 0.0)
        inter = iw * ih
        iou = inter / (area_d + area_g - inter + 1e-8)
        if j == 0:
            best = iou
            bl = jnp.full_like(iou, gl)
            bt = jnp.full_like(iou, gtp)
            br = jnp.full_like(iou, gr)
            bb = jnp.full_like(iou, gb)
            blab = jnp.full_like(iou, lab)
        else:
            upd = iou > best
            best = jnp.where(upd, iou, best)
            bl = jnp.where(upd, gl, bl)
            bt = jnp.where(upd, gtp, bt)
            br = jnp.where(upd, gr, br)
            bb = jnp.where(upd, gb, bb)
            blab = jnp.where(upd, lab, blab)

    mask = best > THR
    maskf = mask.astype(jnp.float32)
    pos_b = jnp.sum(maskf)

    pl_ = plt_ref[0]
    sl1 = jnp.zeros_like(best)
    for c, bc in enumerate((bl, bt, br, bb)):
        dd = pl_[c] - bc
        adx = jnp.abs(dd)
        sl1 = sl1 + jnp.where(adx < 1.0, 0.5 * dd * dd, adx - 0.5)
    loss_l_b = jnp.sum(maskf * sl1)

    mk_ref[0] = maskf
    ml_ref[0] = jnp.where(mask, blab, 0.0)
    part_ref[0, 0, 0] = pos_b
    part_ref[0, 0, 1] = loss_l_b
    part_ref[0, 0, 2] = 0.0
    part_ref[0, 0, 3] = 0.0


def _ce_body(plabel_ref, ml_ref, se_ref, pk_ref):
    x = plabel_ref[0]
    mlab_i = ml_ref[0].astype(jnp.int32)
    cls_lane = lax.broadcasted_iota(jnp.int32, (D, C), 1)
    ex = jnp.exp(x)
    xs = jnp.where(cls_lane == mlab_i, x, 0.0)
    ones = jnp.ones((C, 1), dtype=jnp.float32)
    se_ref[0] = lax.dot_general(ex, ones, (((1,), (0,)), ((), ())),
                                preferred_element_type=jnp.float32)
    pk_ref[0] = lax.dot_general(xs, ones, (((1,), (0,)), ((), ())),
                                preferred_element_type=jnp.float32)


def _premine_body(se_ref, pk_ref, mf_ref, cn_out_ref, pp_ref):
    se = se_ref[...]
    lse = jnp.log(se)
    loss_c = lse - pk_ref[...]
    mf = mf_ref[...]
    cn_out_ref[...] = (1.0 - mf) * loss_c
    pp_ref[0, 0, 0] = jnp.sum(mf * loss_c)


def _sc_hist_body(cn_hbm, out_hbm, chunk_v, idx_v, ones_v, zeros_v, hist_sp):
    cid = lax.axis_index("c")
    sid = lax.axis_index("s")
    wid = sid * 2 + cid
    pltpu.sync_copy(cn_hbm.at[pl.ds(wid * CHUNK, CHUNK)], chunk_v)

    def fill_step(i, carry):
        ones_v[pl.ds(i * 16, 16)] = jnp.ones((16,), jnp.float32)
        return carry

    lax.fori_loop(0, 8, fill_step, 0)

    def zero_step(i, carry):
        zeros_v[pl.ds(i * 16, 16)] = jnp.zeros((16,), jnp.float32)
        return carry

    lax.fori_loop(0, NBINS // 16, zero_step, 0)

    @pl.when(sid == 0)
    def _zero_hist():
        pltpu.sync_copy(zeros_v, hist_sp)

    plsc.subcore_barrier()

    def scat_step(r, carry):
        for kk in range(8):
            v = chunk_v[pl.ds(r * 128 + kk * 16, 16)]
            bits = lax.bitcast_convert_type(v, jnp.int32)
            idx_v[pl.ds(kk * 16, 16)] = lax.shift_right_logical(
                bits, BIN_SHIFT)
        pltpu.sync_copy(ones_v, hist_sp.at[idx_v], add=True)
        return carry

    lax.fori_loop(0, NSL, scat_step, 0)
    plsc.subcore_barrier()

    @pl.when(sid == 0)
    def _dump():
        pltpu.sync_copy(hist_sp, out_hbm.at[cid])


def _finalize_body(hist_ref, cn_ref, part_ref, pp_ref, out_ref, cnb_ref):
    part = part_ref[...]
    pos = jnp.sum(part[:, :, 0:1])
    loss_l = jnp.sum(part[:, :, 1:2])
    pos_ce = pp_ref[0, 0, 0]
    k_i = (3.0 * pos).astype(jnp.int32)

    h = hist_ref[...]
    hc = jnp.sum(h, axis=0)
    bin_iota = lax.iota(jnp.int32, NBINS)
    k_f = k_i.astype(jnp.float32)

    def bin_step(i, bb):
        cand = bb | (jnp.int32(1) << (jnp.int32(11) - i))
        cnt = jnp.sum(jnp.where(bin_iota >= cand, hc, 0.0))
        return jnp.where(cnt >= k_f, cand, bb)

    beta = lax.fori_loop(0, 12, bin_step, jnp.int32(0))
    t0 = beta << BIN_SHIFT

    cnb_ref[...] = lax.bitcast_convert_type(cn_ref[...], jnp.int32)

    def search_step(i, t):
        cand = t | (jnp.int32(1) << (jnp.int32(BIN_SHIFT - 1) - i))
        cnt = jnp.sum((cnb_ref[...] >= cand).astype(jnp.int32))
        return jnp.where(cnt >= k_i, cand, t)

    t_bits = lax.fori_loop(0, BIN_SHIFT, search_step, t0)

    cnv = cn_ref[...]
    bits = cnb_ref[...]
    gt_m = bits > t_bits
    eq_m = bits == t_bits
    cnt_gt = jnp.sum(gt_m.astype(jnp.int32))
    s_gt = jnp.sum(jnp.where(gt_m, cnv, 0.0))
    cnt_eq = jnp.sum(eq_m.astype(jnp.int32))
    s_eq = jnp.sum(jnp.where(eq_m, cnv, 0.0))
    tval = s_eq / jnp.maximum(cnt_eq, 1).astype(jnp.float32)
    n_tie = jnp.clip(k_i - cnt_gt, 0, cnt_eq).astype(jnp.float32)
    neg_sum = s_gt + n_tie * tval
    neg_sum = jnp.where(k_i >= 1, neg_sum, 0.0)

    total = (loss_l + pos_ce + neg_sum) / jnp.maximum(pos, 1.0)
    out_ref[...] = jnp.full((1, 1), total, dtype=jnp.float32)


def kernel(ploc, plabel, gtloc, gtlabel, dboxes):
    f32 = jnp.float32
    dbt = jnp.pad(dboxes.T, ((0, 0), (0, DP - D))).reshape(4, R, 128)
    plt = jnp.pad(jnp.transpose(ploc, (0, 2, 1)),
                  ((0, 0), (0, 0), (0, DP - D))).reshape(B, 4, R, 128)
    g = jnp.concatenate(
        [jnp.transpose(gtloc, (0, 2, 1)),
         gtlabel.astype(f32)[:, None, :]], axis=1).reshape(B, 1, 5, NG)

    mk, ml, part = pl.pallas_call(
        _match_body,
        grid=(B,),
        in_specs=[
            pl.BlockSpec((4, R, 128), lambda b: (0, 0, 0)),
            pl.BlockSpec((1, 4, R, 128), lambda b: (b, 0, 0, 0)),
            pl.BlockSpec((1, 1, 5, NG), lambda b: (b, 0, 0, 0),
                         memory_space=pltpu.SMEM),
        ],
        out_specs=[
            pl.BlockSpec((1, R, 128), lambda b: (b, 0, 0)),
            pl.BlockSpec((1, R, 128), lambda b: (b, 0, 0)),
            pl.BlockSpec((1, 1, 4), lambda b: (b, 0, 0),
                         memory_space=pltpu.SMEM),
        ],
        out_shape=[
            jax.ShapeDtypeStruct((B, R, 128), f32),
            jax.ShapeDtypeStruct((B, R, 128), f32),
            jax.ShapeDtypeStruct((B, 1, 4), f32),
        ],
        compiler_params=pltpu.CompilerParams(
            dimension_semantics=("arbitrary",),
        ),
    )(dbt, plt, g)

    mk_dm = mk.reshape(B, DP)[:, :D, None]
    ml_dm = ml.reshape(B, DP)[:, :D, None]

    se, pk = pl.pallas_call(
        _ce_body,
        grid=(B,),
        in_specs=[
            pl.BlockSpec((1, D, C), lambda b: (b, 0, 0)),
            pl.BlockSpec((1, D, 1), lambda b: (b, 0, 0)),
        ],
        out_specs=[
            pl.BlockSpec((1, D, 1), lambda b: (b, 0, 0)),
            pl.BlockSpec((1, D, 1), lambda b: (b, 0, 0)),
        ],
        out_shape=[
            jax.ShapeDtypeStruct((B, D, 1), f32),
            jax.ShapeDtypeStruct((B, D, 1), f32),
        ],
        compiler_params=pltpu.CompilerParams(
            dimension_semantics=("arbitrary",),
        ),
    )(plabel, ml_dm)

    se_p = se.reshape(PR, 128)
    pk_p = pk.reshape(PR, 128)
    mf_p = mk_dm.reshape(PR, 128)

    cn, pp = pl.pallas_call(
        _premine_body,
        out_specs=[
            pl.BlockSpec((PR, 128), lambda: (0, 0)),
            pl.BlockSpec((1, 1, 4), lambda: (0, 0, 0),
                         memory_space=pltpu.SMEM),
        ],
        out_shape=[
            jax.ShapeDtypeStruct((PR, 128), f32),
            jax.ShapeDtypeStruct((1, 1, 4), f32),
        ],
    )(se_p, pk_p, mf_p)

    cn_flat = jnp.pad(cn.reshape(-1), (0, NPAD - B * D))

    sc_hist = pl.kernel(
        _sc_hist_body,
        out_type=jax.ShapeDtypeStruct((2, NBINS), jnp.float32),
        mesh=plsc.VectorSubcoreMesh(core_axis_name="c",
                                    subcore_axis_name="s"),
        scratch_types=[
            pltpu.VMEM((CHUNK,), f32),
            pltpu.VMEM((128,), jnp.int32),
            pltpu.VMEM((128,), f32),
            pltpu.VMEM((NBINS,), f32),
            pltpu.VMEM_SHARED((NBINS,), f32),
        ],
    )
    hist = sc_hist(cn_flat)

    out = pl.pallas_call(
        _finalize_body,
        out_shape=jax.ShapeDtypeStruct((1, 1), f32),
        scratch_shapes=[
            pltpu.VMEM((PR, 128), jnp.int32),
        ],
    )(hist, cn, part, pp)
    return out[0, 0]
